# baseline (speedup 1.0000x reference)
import jax
import jax.numpy as jnp
from jax import lax
from jax.experimental import pallas as pl
from jax.experimental.pallas import tpu as pltpu
from jax.experimental.pallas import tpu_sc as plsc

NC = 2
NS = 16
L = 16
NW = NC * NS

B = 16384
D = 64
D2 = 2 * D
BPW = B // NW
PASS = BPW // 2
CHUNK = 128
NCH = PASS // CHUNK
GROUPS = PASS // L

NU = 100000
NI = 1000000
NBU = NU // 128
NBI = NI // 128
BPW_U = (NBU + NW - 1) // NW
BPW_I = (NBI + NW - 1) // NW


def _transpose_block(blk, obuf, ncols, lane):
    UQ = 8

    def qq_body(q8, carry):
        for dq in range(UQ):
            qq = q8 * UQ + dq
            for h in range(2):
                col = lane * 0 + (2 * qq + h)
                for k in range(D // L):
                    gv = plsc.load_gather(blk, [k * L + lane, col])
                    obuf[qq, pl.ds(h * D + k * L, L)] = gv
        return carry

    lax.fori_loop(0, ncols // (2 * UQ), qq_body, 0)


def _conv_body(utab_hbm, itab_hbm, tailu_hbm, taili_hbm, cu_hbm, ci_hbm,
               blk_a, blk_b, ob_a, ob_b, sia, sib, soa, sob):
    wid = lax.axis_index("s") * NC + lax.axis_index("c")
    lane = lax.broadcasted_iota(jnp.int32, (L,), 0)

    def stream(tab_hbm, out_hbm, lo, hi, kmax):
        def fetch(c, blk, sem):
            off = pl.multiple_of(c * 128, 128)
            pltpu.make_async_copy(
                tab_hbm.at[:, pl.ds(off, 128)], blk, sem).start()

        def fdrain(blk, sem):
            pltpu.make_async_copy(
                tab_hbm.at[:, pl.ds(0, 128)], blk, sem).wait()

        def wstart(c, ob, sem):
            off = pl.multiple_of(c * 64, 8)
            pltpu.make_async_copy(
                ob, out_hbm.at[pl.ds(off, 64), :], sem).start()

        def wdrain(ob, sem):
            pltpu.make_async_copy(
                ob, out_hbm.at[pl.ds(0, 64), :], sem).wait()

        @pl.when(lo < hi)
        def _():
            fetch(lo, blk_a, sia)

        def body(k2, carry):
            c0 = lo + 2 * k2
            c1 = c0 + 1

            @pl.when(c1 < hi)
            def _():
                fetch(c1, blk_b, sib)

            @pl.when(c0 < hi)
            def _():
                fdrain(blk_a, sia)

                @pl.when(k2 >= 1)
                def _():
                    wdrain(ob_a, soa)

                _transpose_block(blk_a, ob_a, 128, lane)
                wstart(c0, ob_a, soa)

            @pl.when(c0 + 2 < hi)
            def _():
                fetch(c0 + 2, blk_a, sia)

            @pl.when(c1 < hi)
            def _():
                @pl.when(k2 >= 1)
                def _():
                    wdrain(ob_b, sob)

                fdrain(blk_b, sib)
                _transpose_block(blk_b, ob_b, 128, lane)
                wstart(c1, ob_b, sob)

            return carry

        lax.fori_loop(0, kmax, body, 0)

        nb = hi - lo

        @pl.when(nb >= 1)
        def _():
            wdrain(ob_a, soa)

        @pl.when(nb >= 2)
        def _():
            wdrain(ob_b, sob)

    lo_i = wid * BPW_I
    stream(itab_hbm, ci_hbm, lo_i,
           jnp.minimum(lo_i + BPW_I, NBI), (BPW_I + 1) // 2)
    lo_u = wid * BPW_U
    stream(utab_hbm, cu_hbm, lo_u,
           jnp.minimum(lo_u + BPW_U, NBU), (BPW_U + 1) // 2)

    @pl.when(wid == NW - 1)
    def _():
        pltpu.sync_copy(taili_hbm, ob_a.at[pl.ds(0, 32), :])
        pltpu.sync_copy(ob_a.at[pl.ds(0, 32), :],
                        ci_hbm.at[pl.ds(NBI * 64, 32), :])
        pltpu.sync_copy(tailu_hbm, ob_a.at[pl.ds(0, 16), :])
        pltpu.sync_copy(ob_a.at[pl.ds(0, 16), :],
                        cu_hbm.at[pl.ds(NBU * 64, 16), :])


_conv_call = pl.kernel(
    _conv_body,
    out_type=(jax.ShapeDtypeStruct((NU // 2, D2), jnp.float32),
              jax.ShapeDtypeStruct((NI // 2, D2), jnp.float32)),
    mesh=plsc.VectorSubcoreMesh(core_axis_name="c", subcore_axis_name="s"),
    compiler_params=pltpu.CompilerParams(
        needs_layout_passes=False, use_tc_tiling_on_sc=True),
    scratch_types=[
        pltpu.VMEM((D, 128), jnp.float32),
        pltpu.VMEM((D, 128), jnp.float32),
        pltpu.VMEM((64, D2), jnp.float32),
        pltpu.VMEM((64, D2), jnp.float32),
        pltpu.SemaphoreType.DMA,
        pltpu.SemaphoreType.DMA,
        pltpu.SemaphoreType.DMA,
        pltpu.SemaphoreType.DMA,
    ],
)


def _mf_body(uidx_hbm, iidx_hbm, utab_hbm, itab_hbm, out_hbm,
             uidx_v, iidx_v, ug_v, ig_v, upair_v, ipair_v, out_v, sem):
    wid = lax.axis_index("s") * NC + lax.axis_index("c")
    base = wid * BPW

    pltpu.sync_copy(uidx_hbm.at[wid], uidx_v)
    pltpu.sync_copy(iidx_hbm.at[wid], iidx_v)

    def shift_body(i, carry):
        ug_v[pl.ds(i * L, L)] = lax.shift_right_logical(
            uidx_v[pl.ds(i * L, L)], 1)
        ig_v[pl.ds(i * L, L)] = lax.shift_right_logical(
            iidx_v[pl.ds(i * L, L)], 1)
        return carry

    lax.fori_loop(0, BPW // L, shift_body, 0)

    lane = lax.broadcasted_iota(jnp.int32, (L,), 0)
    one = jnp.ones((L,), jnp.int32)

    for p in range(2):
        copies = []
        for c in range(NCH):
            off = p * PASS + c * CHUNK
            copies.append(pltpu.async_copy(
                utab_hbm.at[ug_v.at[pl.ds(off, CHUNK)]],
                upair_v.at[pl.ds(c * CHUNK, CHUNK)], sem))
            copies.append(pltpu.async_copy(
                itab_hbm.at[ig_v.at[pl.ds(off, CHUNK)]],
                ipair_v.at[pl.ds(c * CHUNK, CHUNK)], sem))
        for cp in copies:
            cp.wait()

        def group_body(g, carry):
            row0 = g * L
            ll = jnp.zeros((L,), jnp.float32)
            lh = jnp.zeros((L,), jnp.float32)
            hl = jnp.zeros((L,), jnp.float32)
            hh = jnp.zeros((L,), jnp.float32)
            for r in range(L):
                row = row0 + r
                sll = jnp.zeros((L,), jnp.float32)
                slh = jnp.zeros((L,), jnp.float32)
                shl = jnp.zeros((L,), jnp.float32)
                shh = jnp.zeros((L,), jnp.float32)
                for k in range(D // L):
                    ulo = upair_v[row, pl.ds(k * L, L)]
                    uhi = upair_v[row, pl.ds(D + k * L, L)]
                    ilo = ipair_v[row, pl.ds(k * L, L)]
                    ihi = ipair_v[row, pl.ds(D + k * L, L)]
                    sll = sll + ulo * ilo
                    slh = slh + ulo * ihi
                    shl = shl + uhi * ilo
                    shh = shh + uhi * ihi
                sel = lane == r
                ll = jnp.where(sel, jnp.sum(sll), ll)
                lh = jnp.where(sel, jnp.sum(slh), lh)
                hl = jnp.where(sel, jnp.sum(shl), hl)
                hh = jnp.where(sel, jnp.sum(shh), hh)
            boff = p * PASS + row0
            pu = (uidx_v[pl.ds(boff, L)] & one) == one
            pi = (iidx_v[pl.ds(boff, L)] & one) == one
            out_v[pl.ds(boff, L)] = jnp.where(
                pu, jnp.where(pi, hh, hl), jnp.where(pi, lh, ll))
            return carry

        lax.fori_loop(0, GROUPS, group_body, 0)

    pltpu.sync_copy(out_v, out_hbm.at[pl.ds(base, BPW)])


_mf_call = pl.kernel(
    _mf_body,
    out_type=jax.ShapeDtypeStruct((B,), jnp.float32),
    mesh=plsc.VectorSubcoreMesh(core_axis_name="c", subcore_axis_name="s"),
    compiler_params=pltpu.CompilerParams(
        needs_layout_passes=False, use_tc_tiling_on_sc=True),
    scratch_types=[
        pltpu.VMEM((BPW,), jnp.int32),
        pltpu.VMEM((BPW,), jnp.int32),
        pltpu.VMEM((BPW,), jnp.int32),
        pltpu.VMEM((BPW,), jnp.int32),
        pltpu.VMEM((PASS, D2), jnp.float32),
        pltpu.VMEM((PASS, D2), jnp.float32),
        pltpu.VMEM((BPW,), jnp.float32),
        pltpu.SemaphoreType.DMA,
    ],
)


@jax.jit
def kernel(user_indices, item_indices, user_table, item_table):
    uidx = user_indices.astype(jnp.int32).reshape(NW, BPW)
    iidx = item_indices.astype(jnp.int32).reshape(NW, BPW)
    tail_u = user_table[NBU * 128:NU].reshape(16, D2)
    tail_i = item_table[NBI * 128:NI].reshape(32, D2)
    cu, ci = _conv_call(user_table.T, item_table.T, tail_u, tail_i)
    return _mf_call(uidx, iidx, cu, ci)


# speedup vs baseline: 3.7913x; 3.7913x over previous
import jax
import jax.numpy as jnp
from jax import lax
from jax.experimental import pallas as pl
from jax.experimental.pallas import tpu as pltpu
from jax.experimental.pallas import tpu_sc as plsc

NC = 2
NS = 16
L = 16
NW = NC * NS

B = 16384
D = 64
BPW = B // NW
PE = 16
NP = BPW // PE


def _mf_body(uidx_hbm, iidx_hbm, utab_hbm, itab_hbm, out_hbm,
             uidx_v, iidx_v, ua_v, ia_v, ub_v, ib_v, out_v, sema, semb):
    wid = lax.axis_index("s") * NC + lax.axis_index("c")
    base = wid * BPW

    pltpu.sync_copy(uidx_hbm.at[wid], uidx_v)
    pltpu.sync_copy(iidx_hbm.at[wid], iidx_v)

    lane = lax.broadcasted_iota(jnp.int32, (L,), 0)

    def issue(p, ubuf, ibuf, sem):
        vecu = uidx_v[0, pl.ds(p * PE, L)]
        veci = iidx_v[0, pl.ds(p * PE, L)]
        for r in range(L):
            ub = pl.multiple_of((vecu[r] >> 3) * 8, 8)
            ib = pl.multiple_of((veci[r] >> 3) * 8, 8)
            pltpu.make_async_copy(
                utab_hbm.at[pl.ds(ub, 8), :],
                ubuf.at[pl.ds(r * 8, 8), :], sem).start()
            pltpu.make_async_copy(
                itab_hbm.at[pl.ds(ib, 8), :],
                ibuf.at[pl.ds(r * 8, 8), :], sem).start()

    def drain(ubuf, ibuf, sem):
        pltpu.make_async_copy(
            utab_hbm.at[pl.ds(0, PE * 8), :], ubuf, sem).wait()
        pltpu.make_async_copy(
            itab_hbm.at[pl.ds(0, PE * 8), :], ibuf, sem).wait()

    def compute(p, ubuf, ibuf):
        vecu = uidx_v[0, pl.ds(p * PE, L)]
        veci = iidx_v[0, pl.ds(p * PE, L)]
        vec = jnp.zeros((L,), jnp.float32)
        for r in range(L):
            su = r * 8 + (vecu[r] & 7)
            si = r * 8 + (veci[r] & 7)
            acc = ubuf[su, pl.ds(0, L)] * ibuf[si, pl.ds(0, L)]
            for k in range(1, D // L):
                acc = acc + (ubuf[su, pl.ds(k * L, L)]
                             * ibuf[si, pl.ds(k * L, L)])
            vec = jnp.where(lane == r, jnp.sum(acc), vec)
        out_v[pl.ds(p * PE, L)] = vec

    issue(0, ua_v, ia_v, sema)

    def pipe_body(k, carry):
        p = k * 2
        issue(p + 1, ub_v, ib_v, semb)
        drain(ua_v, ia_v, sema)
        compute(p, ua_v, ia_v)

        @pl.when(k < NP // 2 - 1)
        def _():
            issue(p + 2, ua_v, ia_v, sema)

        drain(ub_v, ib_v, semb)
        compute(p + 1, ub_v, ib_v)
        return carry

    lax.fori_loop(0, NP // 2, pipe_body, 0)

    pltpu.sync_copy(out_v, out_hbm.at[pl.ds(base, BPW)])


_mf_call = pl.kernel(
    _mf_body,
    out_type=jax.ShapeDtypeStruct((B,), jnp.float32),
    mesh=plsc.VectorSubcoreMesh(core_axis_name="c", subcore_axis_name="s"),
    compiler_params=pltpu.CompilerParams(
        needs_layout_passes=False, use_tc_tiling_on_sc=True),
    scratch_types=[
        pltpu.VMEM((1, BPW), jnp.int32),
        pltpu.VMEM((1, BPW), jnp.int32),
        pltpu.VMEM((PE * 8, D), jnp.float32),
        pltpu.VMEM((PE * 8, D), jnp.float32),
        pltpu.VMEM((PE * 8, D), jnp.float32),
        pltpu.VMEM((PE * 8, D), jnp.float32),
        pltpu.VMEM((BPW,), jnp.float32),
        pltpu.SemaphoreType.DMA,
        pltpu.SemaphoreType.DMA,
    ],
)


@jax.jit
def kernel(user_indices, item_indices, user_table, item_table):
    uidx = user_indices.astype(jnp.int32).reshape(NW, 1, BPW)
    iidx = item_indices.astype(jnp.int32).reshape(NW, 1, BPW)
    return _mf_call(uidx, iidx, user_table, item_table)
